# Initial kernel scaffold; baseline (speedup 1.0000x reference)
import jax
import jax.numpy as jnp
from jax.experimental import pallas as pl


def kernel(node_features, edge_index, edge_features, W, b):
    raise NotImplementedError("write your pallas kernel here")


# speedup vs baseline: 2.8727x; 2.8727x over previous
import functools

import jax
import jax.numpy as jnp
from jax import lax
from jax.experimental import pallas as pl
from jax.experimental.pallas import tpu as pltpu
from jax.experimental.pallas import tpu_sc as plsc

_NODE_DIM = 128
_EDGE_DIM = 16
_N_NODES = 10000
_N_EDGES = 320000

_NC = 2
_NS = 16
_NW = _NC * _NS
_BPW = _N_EDGES // _NW
_C = 2000
_NCHUNK = _BPW // _C

_EB = 8000
_NEB = _N_EDGES // _EB


def _proj_body(nf_ref, w_ref, p1_ref, p2_ref):
    nf = nf_ref[...]
    w = w_ref[...]
    p1_ref[...] = jnp.dot(nf, w[:_NODE_DIM], preferred_element_type=jnp.float32)
    p2_ref[...] = jnp.dot(nf, w[_NODE_DIM:], preferred_element_type=jnp.float32)


def _node_proj(nf, w12):
    return pl.pallas_call(
        _proj_body,
        out_shape=[
            jax.ShapeDtypeStruct((_N_NODES, _EDGE_DIM), jnp.float32),
            jax.ShapeDtypeStruct((_N_NODES, _EDGE_DIM), jnp.float32),
        ],
    )(nf, w12)


def _sc_gather_body(p1_hbm, p2_hbm, row_hbm, col_hbm, g1_hbm, g2_hbm,
                    idx1, idx2, r1, r2, sem1, sem2):
    cid = lax.axis_index("c")
    sid = lax.axis_index("s")
    wid = sid * _NC + cid
    base0 = wid * _BPW
    for i in range(_NCHUNK):
        base = base0 + i * _C
        pltpu.sync_copy(row_hbm.at[pl.ds(base, _C)], idx1)
        pltpu.sync_copy(col_hbm.at[pl.ds(base, _C)], idx2)
        c1 = pltpu.async_copy(p1_hbm.at[idx1], r1, sem1)
        c2 = pltpu.async_copy(p2_hbm.at[idx2], r2, sem2)
        c1.wait()
        c2.wait()
        pltpu.sync_copy(r1, g1_hbm.at[pl.ds(base, _C)])
        pltpu.sync_copy(r2, g2_hbm.at[pl.ds(base, _C)])


_sc_gather = functools.partial(
    pl.kernel,
    mesh=plsc.VectorSubcoreMesh(core_axis_name="c", subcore_axis_name="s"),
    out_type=[
        jax.ShapeDtypeStruct((_N_EDGES, _EDGE_DIM), jnp.float32),
        jax.ShapeDtypeStruct((_N_EDGES, _EDGE_DIM), jnp.float32),
    ],
    compiler_params=pltpu.CompilerParams(use_tc_tiling_on_sc=False),
    scratch_types=[
        pltpu.VMEM((_C,), jnp.int32),
        pltpu.VMEM((_C,), jnp.int32),
        pltpu.VMEM((_C, _EDGE_DIM), jnp.float32),
        pltpu.VMEM((_C, _EDGE_DIM), jnp.float32),
        pltpu.SemaphoreType.DMA,
        pltpu.SemaphoreType.DMA,
    ],
)(_sc_gather_body)


def _edge_body(ef_ref, g1_ref, g2_ref, w3_ref, b_ref, out_ref):
    ef = ef_ref[...]
    x = (g1_ref[...] + g2_ref[...] + b_ref[...]
         + jnp.dot(ef, w3_ref[...], preferred_element_type=jnp.float32))
    out_ref[...] = ef * jax.nn.sigmoid(x)


def _edge_update(ef, g1, g2, w3, b2):
    eb_spec = pl.BlockSpec((_EB, _EDGE_DIM), lambda i: (i, 0))
    return pl.pallas_call(
        _edge_body,
        grid=(_NEB,),
        in_specs=[
            eb_spec,
            eb_spec,
            eb_spec,
            pl.BlockSpec((_EDGE_DIM, _EDGE_DIM), lambda i: (0, 0)),
            pl.BlockSpec((1, _EDGE_DIM), lambda i: (0, 0)),
        ],
        out_specs=eb_spec,
        out_shape=jax.ShapeDtypeStruct((_N_EDGES, _EDGE_DIM), jnp.float32),
    )(ef, g1, g2, w3, b2)


def kernel(node_features, edge_index, edge_features, W, b):
    ei = edge_index.astype(jnp.int32)
    row, col = ei[0], ei[1]
    p1, p2 = _node_proj(node_features, W[: 2 * _NODE_DIM])
    g1, g2 = _sc_gather(p1, p2, row, col)
    return _edge_update(edge_features, g1, g2, W[2 * _NODE_DIM:],
                        b.reshape(1, _EDGE_DIM))


# speedup vs baseline: 4.8556x; 1.6903x over previous
import functools

import jax
import jax.numpy as jnp
from jax import lax
from jax.experimental import pallas as pl
from jax.experimental.pallas import tpu as pltpu
from jax.experimental.pallas import tpu_sc as plsc

_NODE_DIM = 128
_EDGE_DIM = 16
_N_NODES = 10000
_N_EDGES = 320000

_NC = 2
_NS = 16
_NW = _NC * _NS
_BPW = _N_EDGES // _NW
_C = 2000
_NCHUNK = _BPW // _C

_EB = 6400
_NEB = _N_EDGES // _EB


def _proj_body(nf_ref, w_ref, p1_ref, p2_ref):
    nf = nf_ref[...]
    w = w_ref[...]
    p1_ref[...] = jnp.dot(nf, w[:_NODE_DIM], preferred_element_type=jnp.float32)
    p2_ref[...] = jnp.dot(nf, w[_NODE_DIM:], preferred_element_type=jnp.float32)


def _node_proj(nf, w12):
    return pl.pallas_call(
        _proj_body,
        out_shape=[
            jax.ShapeDtypeStruct((_N_NODES, _EDGE_DIM), jnp.float32),
            jax.ShapeDtypeStruct((_N_NODES, _EDGE_DIM), jnp.float32),
        ],
    )(nf, w12)


def _sc_gather_body(p1_hbm, p2_hbm, row_hbm, col_hbm, g1_hbm, g2_hbm,
                    idx1, idx2, r1, r2, sem1, sem2):
    cid = lax.axis_index("c")
    sid = lax.axis_index("s")
    wid = sid * _NC + cid
    base0 = wid * _BPW
    for i in range(_NCHUNK):
        base = base0 + i * _C
        pltpu.sync_copy(row_hbm.at[pl.ds(base, _C)], idx1)
        pltpu.sync_copy(col_hbm.at[pl.ds(base, _C)], idx2)
        c1 = pltpu.async_copy(p1_hbm.at[idx1], r1, sem1)
        c2 = pltpu.async_copy(p2_hbm.at[idx2], r2, sem2)
        c1.wait()
        c2.wait()
        pltpu.sync_copy(r1, g1_hbm.at[pl.ds(base, _C)])
        pltpu.sync_copy(r2, g2_hbm.at[pl.ds(base, _C)])


_sc_gather = functools.partial(
    pl.kernel,
    mesh=plsc.VectorSubcoreMesh(core_axis_name="c", subcore_axis_name="s"),
    out_type=[
        jax.ShapeDtypeStruct((_N_EDGES, _EDGE_DIM), jnp.float32),
        jax.ShapeDtypeStruct((_N_EDGES, _EDGE_DIM), jnp.float32),
    ],
    compiler_params=pltpu.CompilerParams(use_tc_tiling_on_sc=False),
    scratch_types=[
        pltpu.VMEM((_C,), jnp.int32),
        pltpu.VMEM((_C,), jnp.int32),
        pltpu.VMEM((_C, _EDGE_DIM), jnp.float32),
        pltpu.VMEM((_C, _EDGE_DIM), jnp.float32),
        pltpu.SemaphoreType.DMA,
        pltpu.SemaphoreType.DMA,
    ],
)(_sc_gather_body)


def _edge_body(ef_ref, g1_ref, g2_ref, w3b_ref, b_ref, out_ref):
    ef = ef_ref[...]
    x = (g1_ref[...] + g2_ref[...] + b_ref[...]
         + jnp.dot(ef, w3b_ref[...], preferred_element_type=jnp.float32))
    out_ref[...] = ef * jax.nn.sigmoid(x)


def _edge_update(ef128, g1p, g2p, w3b, b128):
    gb_spec = pl.BlockSpec((_EB // 8, 8 * _EDGE_DIM), lambda i: (i, 0))
    return pl.pallas_call(
        _edge_body,
        grid=(_NEB,),
        in_specs=[
            gb_spec,
            gb_spec,
            gb_spec,
            pl.BlockSpec((8 * _EDGE_DIM, 8 * _EDGE_DIM), lambda i: (0, 0)),
            pl.BlockSpec((1, 8 * _EDGE_DIM), lambda i: (0, 0)),
        ],
        out_specs=gb_spec,
        out_shape=jax.ShapeDtypeStruct((_N_EDGES // 8, 8 * _EDGE_DIM),
                                       jnp.float32),
    )(ef128, g1p, g2p, w3b, b128)


def kernel(node_features, edge_index, edge_features, W, b):
    ei = edge_index.astype(jnp.int32)
    row, col = ei[0], ei[1]
    p1, p2 = _node_proj(node_features, W[: 2 * _NODE_DIM])
    g1, g2 = _sc_gather(p1, p2, row, col)
    ef128 = jnp.reshape(edge_features, (_N_EDGES // 8, 8 * _EDGE_DIM))
    g1p = jnp.reshape(g1, (_N_EDGES // 8, 8 * _EDGE_DIM))
    g2p = jnp.reshape(g2, (_N_EDGES // 8, 8 * _EDGE_DIM))
    w3b = jnp.kron(jnp.eye(8, dtype=jnp.float32), W[2 * _NODE_DIM:])
    b128 = jnp.tile(b, 8).reshape(1, 8 * _EDGE_DIM)
    out128 = _edge_update(ef128, g1p, g2p, w3b, b128)
    return jnp.reshape(out128, (_N_EDGES, _EDGE_DIM))


# speedup vs baseline: 7.3156x; 1.5066x over previous
import functools

import jax
import jax.numpy as jnp
from jax import lax
from jax.experimental import pallas as pl
from jax.experimental.pallas import tpu as pltpu
from jax.experimental.pallas import tpu_sc as plsc

_NODE_DIM = 128
_EDGE_DIM = 16
_N_NODES = 10000
_N_EDGES = 320000

_NC = 2
_NS = 16
_NW = _NC * _NS
_BPW = _N_EDGES // _NW
_C = 2000
_NCHUNK = _BPW // _C

_EB = 16000
_PIECE = _EB // 8
_NEB = _N_EDGES // _EB


def _proj_body(nf_ref, w_ref, p1_ref, p2_ref):
    nf = nf_ref[...]
    w = w_ref[...]
    p1_ref[...] = jnp.dot(nf, w[:_NODE_DIM], preferred_element_type=jnp.float32)
    p2_ref[...] = jnp.dot(nf, w[_NODE_DIM:], preferred_element_type=jnp.float32)


def _node_proj(nf, w12):
    return pl.pallas_call(
        _proj_body,
        out_shape=[
            jax.ShapeDtypeStruct((_N_NODES, _EDGE_DIM), jnp.float32),
            jax.ShapeDtypeStruct((_N_NODES, _EDGE_DIM), jnp.float32),
        ],
    )(nf, w12)


def _store_packed(r1, s_hbm, base):
    for j in range(_C // _PIECE):
        e0 = base + j * _PIECE
        rrow = (e0 // _EB) * _PIECE
        kcol = ((e0 % _EB) // _PIECE) * _EDGE_DIM
        pltpu.sync_copy(
            r1.at[pl.ds(j * _PIECE, _PIECE), :],
            s_hbm.at[pl.ds(rrow, _PIECE), pl.ds(kcol, _EDGE_DIM)])


def _sc_gather_body(p1_hbm, p2_hbm, row_hbm, col_hbm, s_hbm,
                    idx1a, idx2a, r1a, idx1b, idx2b, r1b,
                    sem1a, sem2a, sem1b, sem2b):
    cid = lax.axis_index("c")
    sid = lax.axis_index("s")
    wid = sid * _NC + cid
    base0 = wid * _BPW
    bufs = [(idx1a, idx2a, r1a, sem1a, sem2a),
            (idx1b, idx2b, r1b, sem1b, sem2b)]

    idx1, idx2, r1, sem1, _ = bufs[0]
    pltpu.sync_copy(row_hbm.at[pl.ds(base0, _C)], idx1)
    pltpu.sync_copy(col_hbm.at[pl.ds(base0, _C)], idx2)
    g1 = [None] * _NCHUNK
    g1[0] = pltpu.async_copy(p1_hbm.at[idx1], r1, sem1)
    for i in range(_NCHUNK):
        idx1, idx2, r1, sem1, sem2 = bufs[i % 2]
        base = base0 + i * _C
        if i + 1 < _NCHUNK:
            nidx1, nidx2, nr1, nsem1, _ = bufs[(i + 1) % 2]
            pltpu.sync_copy(row_hbm.at[pl.ds(base + _C, _C)], nidx1)
            pltpu.sync_copy(col_hbm.at[pl.ds(base + _C, _C)], nidx2)
        g1[i].wait()
        g2 = pltpu.async_copy(p2_hbm.at[idx2], r1, sem2, add=True)
        if i + 1 < _NCHUNK:
            g1[i + 1] = pltpu.async_copy(p1_hbm.at[nidx1], nr1, nsem1)
        g2.wait()
        _store_packed(r1, s_hbm, base)


_sc_gather = functools.partial(
    pl.kernel,
    mesh=plsc.VectorSubcoreMesh(core_axis_name="c", subcore_axis_name="s"),
    out_type=jax.ShapeDtypeStruct((_N_EDGES // 8, 8 * _EDGE_DIM),
                                  jnp.float32),
    compiler_params=pltpu.CompilerParams(use_tc_tiling_on_sc=False),
    scratch_types=[
        pltpu.VMEM((_C,), jnp.int32),
        pltpu.VMEM((_C,), jnp.int32),
        pltpu.VMEM((_C, _EDGE_DIM), jnp.float32),
        pltpu.VMEM((_C,), jnp.int32),
        pltpu.VMEM((_C,), jnp.int32),
        pltpu.VMEM((_C, _EDGE_DIM), jnp.float32),
        pltpu.SemaphoreType.DMA,
        pltpu.SemaphoreType.DMA,
        pltpu.SemaphoreType.DMA,
        pltpu.SemaphoreType.DMA,
    ],
)(_sc_gather_body)


def _edge_body(efT_ref, sp_ref, w3t_ref, b_ref, outT_ref):
    efT = efT_ref[...]
    sp = sp_ref[...]
    eye = jnp.eye(_EDGE_DIM, dtype=jnp.float32)
    sT = jnp.concatenate(
        [lax.dot_general(eye, sp[:, k * _EDGE_DIM:(k + 1) * _EDGE_DIM],
                         (((1,), (1,)), ((), ())),
                         precision=lax.Precision.HIGHEST,
                         preferred_element_type=jnp.float32)
         for k in range(8)],
        axis=1)
    x = (sT + b_ref[...]
         + jnp.dot(w3t_ref[...], efT, preferred_element_type=jnp.float32))
    outT_ref[...] = efT * jax.nn.sigmoid(x)


def _edge_update(efT, sp, w3t, bcol):
    tb_spec = pl.BlockSpec((_EDGE_DIM, _EB), lambda i: (0, i))
    return pl.pallas_call(
        _edge_body,
        grid=(_NEB,),
        in_specs=[
            tb_spec,
            pl.BlockSpec((_EB // 8, 8 * _EDGE_DIM), lambda i: (i, 0)),
            pl.BlockSpec((_EDGE_DIM, _EDGE_DIM), lambda i: (0, 0)),
            pl.BlockSpec((_EDGE_DIM, 1), lambda i: (0, 0)),
        ],
        out_specs=tb_spec,
        out_shape=jax.ShapeDtypeStruct((_EDGE_DIM, _N_EDGES), jnp.float32),
    )(efT, sp, w3t, bcol)


def kernel(node_features, edge_index, edge_features, W, b):
    ei = edge_index.astype(jnp.int32)
    row, col = ei[0], ei[1]
    p1, p2 = _node_proj(node_features, W[: 2 * _NODE_DIM])
    sp = _sc_gather(p1, p2, row, col)
    outT = _edge_update(edge_features.T, sp, W[2 * _NODE_DIM:].T,
                        b.reshape(_EDGE_DIM, 1))
    return outT.T


# speedup vs baseline: 14.8629x; 2.0317x over previous
import functools

import jax
import jax.numpy as jnp
from jax import lax
from jax.experimental import pallas as pl
from jax.experimental.pallas import tpu as pltpu
from jax.experimental.pallas import tpu_sc as plsc

_NODE_DIM = 128
_EDGE_DIM = 16
_N_NODES = 10000
_N_EDGES = 320000

_NC = 2
_NS = 16
_NW = _NC * _NS
_BPW = _N_EDGES // _NW
_C = 2000
_NCHUNK = _BPW // _C

_EB = 16000
_PIECE = _EB // 8
_NEB = _N_EDGES // _EB


def _proj_body(nf_ref, w_ref, p1_ref, p2_ref):
    nf = nf_ref[...]
    w = w_ref[...]
    p1_ref[...] = jnp.dot(nf, w[:_NODE_DIM], preferred_element_type=jnp.float32)
    p2_ref[...] = jnp.dot(nf, w[_NODE_DIM:], preferred_element_type=jnp.float32)


def _node_proj(nf, w12):
    return pl.pallas_call(
        _proj_body,
        out_shape=[
            jax.ShapeDtypeStruct((_N_NODES, _EDGE_DIM), jnp.float32),
            jax.ShapeDtypeStruct((_N_NODES, _EDGE_DIM), jnp.float32),
        ],
    )(nf, w12)


def _store_packed(r1, s_hbm, base):
    for j in range(_C // _PIECE):
        e0 = base + j * _PIECE
        rrow = (e0 // _EB) * _PIECE
        kcol = ((e0 % _EB) // _PIECE) * _EDGE_DIM
        pltpu.sync_copy(
            r1.at[pl.ds(j * _PIECE, _PIECE), :],
            s_hbm.at[pl.ds(rrow, _PIECE), pl.ds(kcol, _EDGE_DIM)])


def _sc_gather_body(p1_hbm, p2_hbm, row_hbm, col_hbm, s_hbm,
                    t1, t2, idx1a, idx2a, r1a, idx1b, idx2b, r1b,
                    sem1a, sem2a, sem1b, sem2b):
    cid = lax.axis_index("c")
    sid = lax.axis_index("s")
    wid = sid * _NC + cid
    base0 = wid * _BPW

    @pl.when(sid == 0)
    def _():
        pltpu.sync_copy(p1_hbm, t1)
        pltpu.sync_copy(p2_hbm, t2)

    plsc.subcore_barrier()

    bufs = [(idx1a, idx2a, r1a, sem1a, sem2a),
            (idx1b, idx2b, r1b, sem1b, sem2b)]

    idx1, idx2, r1, sem1, _ = bufs[0]
    pltpu.sync_copy(row_hbm.at[pl.ds(base0, _C)], idx1)
    pltpu.sync_copy(col_hbm.at[pl.ds(base0, _C)], idx2)
    g1 = [None] * _NCHUNK
    g1[0] = pltpu.async_copy(t1.at[idx1], r1, sem1)
    for i in range(_NCHUNK):
        idx1, idx2, r1, sem1, sem2 = bufs[i % 2]
        base = base0 + i * _C
        if i + 1 < _NCHUNK:
            nidx1, nidx2, nr1, nsem1, _ = bufs[(i + 1) % 2]
            pltpu.sync_copy(row_hbm.at[pl.ds(base + _C, _C)], nidx1)
            pltpu.sync_copy(col_hbm.at[pl.ds(base + _C, _C)], nidx2)
        g1[i].wait()
        g2 = pltpu.async_copy(t2.at[idx2], r1, sem2, add=True)
        if i + 1 < _NCHUNK:
            g1[i + 1] = pltpu.async_copy(t1.at[nidx1], nr1, nsem1)
        g2.wait()
        _store_packed(r1, s_hbm, base)


_sc_gather = functools.partial(
    pl.kernel,
    mesh=plsc.VectorSubcoreMesh(core_axis_name="c", subcore_axis_name="s"),
    out_type=jax.ShapeDtypeStruct((_N_EDGES // 8, 8 * _EDGE_DIM),
                                  jnp.float32),
    compiler_params=pltpu.CompilerParams(use_tc_tiling_on_sc=False),
    scratch_types=[
        pltpu.VMEM_SHARED((_N_NODES, _EDGE_DIM), jnp.float32),
        pltpu.VMEM_SHARED((_N_NODES, _EDGE_DIM), jnp.float32),
        pltpu.VMEM((_C,), jnp.int32),
        pltpu.VMEM((_C,), jnp.int32),
        pltpu.VMEM((_C, _EDGE_DIM), jnp.float32),
        pltpu.VMEM((_C,), jnp.int32),
        pltpu.VMEM((_C,), jnp.int32),
        pltpu.VMEM((_C, _EDGE_DIM), jnp.float32),
        pltpu.SemaphoreType.DMA,
        pltpu.SemaphoreType.DMA,
        pltpu.SemaphoreType.DMA,
        pltpu.SemaphoreType.DMA,
    ],
)(_sc_gather_body)


def _edge_body(efT_ref, sp_ref, w3t_ref, b_ref, outT_ref):
    efT = efT_ref[...]
    sp = sp_ref[...]
    eye = jnp.eye(_EDGE_DIM, dtype=jnp.float32)
    sT = jnp.concatenate(
        [lax.dot_general(eye, sp[:, k * _EDGE_DIM:(k + 1) * _EDGE_DIM],
                         (((1,), (1,)), ((), ())),
                         preferred_element_type=jnp.float32)
         for k in range(8)],
        axis=1)
    x = (sT + b_ref[...]
         + jnp.dot(w3t_ref[...], efT, preferred_element_type=jnp.float32))
    outT_ref[...] = efT * jax.nn.sigmoid(x)


def _edge_update(efT, sp, w3t, bcol):
    tb_spec = pl.BlockSpec((_EDGE_DIM, _EB), lambda i: (0, i))
    return pl.pallas_call(
        _edge_body,
        grid=(_NEB,),
        in_specs=[
            tb_spec,
            pl.BlockSpec((_EB // 8, 8 * _EDGE_DIM), lambda i: (i, 0)),
            pl.BlockSpec((_EDGE_DIM, _EDGE_DIM), lambda i: (0, 0)),
            pl.BlockSpec((_EDGE_DIM, 1), lambda i: (0, 0)),
        ],
        out_specs=tb_spec,
        out_shape=jax.ShapeDtypeStruct((_EDGE_DIM, _N_EDGES), jnp.float32),
    )(efT, sp, w3t, bcol)


def kernel(node_features, edge_index, edge_features, W, b):
    ei = edge_index.astype(jnp.int32)
    row, col = ei[0], ei[1]
    p1, p2 = _node_proj(node_features, W[: 2 * _NODE_DIM])
    sp = _sc_gather(p1, p2, row, col)
    outT = _edge_update(edge_features.T, sp, W[2 * _NODE_DIM:].T,
                        b.reshape(_EDGE_DIM, 1))
    return outT.T
